# baseline (speedup 1.0000x reference)
import functools

import jax
import jax.numpy as jnp
from jax.experimental import pallas as pl
from jax.experimental.pallas import tpu as pltpu

_BN = 512
_NREL = 14


def _gn(x, g, b, eps=1e-5):
    mu = jnp.mean(x, axis=1, keepdims=True)
    var = jnp.mean((x - mu) ** 2, axis=1, keepdims=True)
    return (x - mu) * jax.lax.rsqrt(var + eps) * g + b


def _enc_body(nd, w0a, b0a, w1a, g1a, t1a, w0b, b0b, w1b, g1b, t1b,
              mw, mg, mt, out):
    nd_ = nd[...]

    def br(x0, x1, W0, b0, W1, g1, bt1):
        h = jnp.maximum(x0 * W0[0:1, :] + x1 * W0[1:2, :] + b0, 0.0)
        return _gn(jnp.dot(h, W1, preferred_element_type=jnp.float32), g1, bt1)

    fa = br(nd_[:, 0:1], nd_[:, 1:2], w0a[...], b0a[...], w1a[...], g1a[...], t1a[...])
    fb = br(nd_[:, 2:3], nd_[:, 3:4], w0b[...], b0b[...], w1b[...], g1b[...], t1b[...])
    f = jnp.maximum(fa + fb, 0.0)
    mw_ = mw[...]
    y = jnp.dot(f, mw_[0:128, :], preferred_element_type=jnp.float32)
    y = y + nd_[:, 4:5] * mw_[128:129, :] + nd_[:, 5:6] * mw_[129:130, :]
    y = y + nd_[:, 6:7] * mw_[130:131, :] + nd_[:, 7:8] * mw_[131:132, :]
    out[...] = jnp.maximum(_gn(y, mg[...], mt[...]), 0.0)


def _layA_body(f, wc, wr, t_out, y_out):
    f_ = f[...]
    t_out[...] = jnp.dot(f_, wc[...], preferred_element_type=jnp.float32)
    y_out[...] = jnp.dot(f_, wr[...], preferred_element_type=jnp.float32)


def _layB_body(t, r, w2, ng, nt, g2, t2, out):
    h = jnp.maximum(_gn(t[...], ng[...], nt[...]), 0.0)
    u = _gn(jnp.dot(h, w2[...], preferred_element_type=jnp.float32), g2[...], t2[...])
    out[...] = jnp.maximum(u + r[...], 0.0)


def _full(shape):
    return pl.BlockSpec(shape, lambda i: (0,) * len(shape))


def _rows(c):
    return pl.BlockSpec((_BN, c), lambda i: (i, 0))


def _encoder(nodes_p, p, npad, c):
    grid = (npad // _BN,)
    w = lambda s: _full(s)
    return pl.pallas_call(
        _enc_body,
        grid=grid,
        in_specs=[_rows(8)] + [w((2, c)), w((1, c)), w((c, c)), w((1, c)), w((1, c))] * 2
        + [w((c + 4, c)), w((1, c)), w((1, c))],
        out_specs=_rows(c),
        out_shape=jax.ShapeDtypeStruct((npad, c), jnp.float32),
    )(nodes_p,
      p['in_W0'], p['in_b0'].reshape(1, c), p['in_W1'],
      p['in_g1'].reshape(1, c), p['in_bt1'].reshape(1, c),
      p['seg_W0'], p['seg_b0'].reshape(1, c), p['seg_W1'],
      p['seg_g1'].reshape(1, c), p['seg_bt1'].reshape(1, c),
      p['meta_W'], p['meta_g'].reshape(1, c), p['meta_bt'].reshape(1, c))


def _layA(feat, wc, wr, npad, c):
    grid = (npad // _BN,)
    return pl.pallas_call(
        _layA_body,
        grid=grid,
        in_specs=[_rows(c), _full((c, c)), _full((c, _NREL * c))],
        out_specs=[_rows(c), _rows(_NREL * c)],
        out_shape=[jax.ShapeDtypeStruct((npad, c), jnp.float32),
                   jax.ShapeDtypeStruct((npad, _NREL * c), jnp.float32)],
    )(feat, wc, wr)


def _layB(temp, res, w2, ng, nt, g2, t2, npad, c):
    grid = (npad // _BN,)
    return pl.pallas_call(
        _layB_body,
        grid=grid,
        in_specs=[_rows(c), _rows(c), _full((c, c))] + [_full((1, c))] * 4,
        out_specs=_rows(c),
        out_shape=jax.ShapeDtypeStruct((npad, c), jnp.float32),
    )(temp, res, w2, ng.reshape(1, c), nt.reshape(1, c),
      g2.reshape(1, c), t2.reshape(1, c))


def _scatter_jnp(temp, y, indexes, mask, n, c):
    e = indexes.shape[0]
    row = jnp.arange(e, dtype=jnp.int32)
    y3 = y[:n].reshape(n, _NREL, c)
    for j in range(_NREL):
        valid = row < mask[j]
        dst = indexes[:, 2 * j]
        src = indexes[:, 2 * j + 1]
        upd = jnp.where(valid[:, None], y3[src, j, :], jnp.float32(0))
        temp = temp.at[dst].add(upd)
    return temp


def kernel(nodes, indexes, mask, params):
    n = nodes.shape[0]
    c = params['in_W1'].shape[0]
    npad = ((n + _BN - 1) // _BN) * _BN

    nodes_p = jnp.pad(nodes, ((0, npad - n), (0, 0)))
    feat = _encoder(nodes_p, params, npad, c)

    res = feat
    for i in range(4):
        wr = jnp.transpose(params['rel_W'][i], (1, 0, 2)).reshape(c, _NREL * c)
        temp, y = _layA(feat, params['ctr_W'][i], wr, npad, c)
        temp = temp.at[:n].set(_scatter_jnp(temp[:n], y, indexes, mask, n, c))
        feat = _layB(temp, res, params['ctr2_W'][i],
                     params['norm_g'][i], params['norm_bt'][i],
                     params['ctr2_g'][i], params['ctr2_bt'][i], npad, c)
        res = feat
    return (feat[:n], nodes[:, :2])
